# baseline (speedup 1.0000x reference)
import functools

import jax
import jax.numpy as jnp
from jax import lax
from jax.experimental import pallas as pl
from jax.experimental.pallas import tpu as pltpu
from jax.experimental.pallas import tpu_sc as plsc

_ALPHA = 0.2
_BETA = 0.1
_CE = 64
_NW = 32

f32 = jnp.float32


def _prep(xp, W_cat, A_src, A_dst):
    NPp, KD = xp.shape
    FD = W_cat.shape[1]
    BN = 1024

    def body(x_ref, w_ref, as_ref, ad_ref, h_ref, s_ref, d_ref):
        h = jnp.dot(x_ref[...], w_ref[...], preferred_element_type=f32)
        h_ref[...] = h
        s_ref[...] = jnp.dot(h, as_ref[...], preferred_element_type=f32)
        d_ref[...] = jnp.dot(h, ad_ref[...], preferred_element_type=f32)

    return pl.pallas_call(
        body,
        grid=(pl.cdiv(NPp, BN),),
        in_specs=[
            pl.BlockSpec((BN, KD), lambda i: (i, 0)),
            pl.BlockSpec((KD, FD), lambda i: (0, 0)),
            pl.BlockSpec((FD, 16), lambda i: (0, 0)),
            pl.BlockSpec((FD, 16), lambda i: (0, 0)),
        ],
        out_specs=[
            pl.BlockSpec((BN, FD), lambda i: (i, 0)),
            pl.BlockSpec((BN, 16), lambda i: (i, 0)),
            pl.BlockSpec((BN, 16), lambda i: (i, 0)),
        ],
        out_shape=[
            jax.ShapeDtypeStruct((NPp, FD), f32),
            jax.ShapeDtypeStruct((NPp, 16), f32),
            jax.ShapeDtypeStruct((NPp, 16), f32),
        ],
    )(xp, W_cat, A_src, A_dst)


def _merge(p, d, h, expand, apply_elu):
    NPp, F = h.shape
    BN = 512

    def body(p_ref, d_ref, h_ref, e_ref, o_ref):
        den = d_ref[0] + d_ref[1] + 1e-16
        inv = jnp.dot(1.0 / den, e_ref[...], preferred_element_type=f32)
        v = (1.0 - _BETA) * (p_ref[0] + p_ref[1]) * inv + _BETA * h_ref[...]
        if apply_elu:
            v = jnp.where(v > 0, v, jnp.exp(v) - 1.0)
        o_ref[...] = v

    return pl.pallas_call(
        body,
        grid=(pl.cdiv(NPp, BN),),
        in_specs=[
            pl.BlockSpec((2, BN, F), lambda i: (0, i, 0)),
            pl.BlockSpec((2, BN, 16), lambda i: (0, i, 0)),
            pl.BlockSpec((BN, F), lambda i: (i, 0)),
            pl.BlockSpec((16, F), lambda i: (0, 0)),
        ],
        out_specs=pl.BlockSpec((BN, F), lambda i: (i, 0)),
        out_shape=jax.ShapeDtypeStruct((NPp, F), f32),
    )(p, d, h, expand)


def _log_softmax(x):
    NPp, C = x.shape
    BN = 2048

    def body(x_ref, o_ref):
        v = x_ref[...]
        m = jnp.max(v, axis=1, keepdims=True)
        ex = jnp.exp(v - m)
        s = jnp.sum(ex, axis=1, keepdims=True)
        o_ref[...] = v - m - jnp.log(s)

    return pl.pallas_call(
        body,
        grid=(pl.cdiv(NPp, BN),),
        in_specs=[pl.BlockSpec((BN, C), lambda i: (i, 0))],
        out_specs=pl.BlockSpec((BN, C), lambda i: (i, 0)),
        out_shape=jax.ShapeDtypeStruct((NPp, C), f32),
    )(x)


def _make_edge_pass(NPp, F, H, Ep):
    B = F // 16
    cpb = F // H
    ZR = NPp // 16
    n_chunks = Ep // (_NW * _CE)
    mesh = plsc.VectorSubcoreMesh(
        core_axis_name="c", subcore_axis_name="s", num_cores=2, num_subcores=16
    )

    def body(src_hbm, dst_hbm, asrc_hbm, adst_hbm, feat_hbm, p_out, d_out,
             sidxA, didxA, arA, adA, frA, exbA, wbA, didxSA,
             sidxB, didxB, arB, adB, frB, exbB, wbB, didxSB,
             p_sh, d_sh, gsemA, gsemB, ssemA, ssemB):
        c = lax.axis_index("c")
        s = lax.axis_index("s")
        wid = s * 2 + c

        zvec = jnp.zeros((16,), f32)
        zivec = jnp.zeros((16,), jnp.int32)

        @pl.loop(0, _CE)
        def _zbuf(r):
            for b in range(B):
                wbA[r, b * 16:(b + 1) * 16] = zvec
                wbB[r, b * 16:(b + 1) * 16] = zvec
            exbA[r, :] = zvec
            exbB[r, :] = zvec

        @pl.loop(0, _CE // 16)
        def _zidx(j):
            didxSA[pl.ds(j * 16, 16)] = zivec
            didxSB[pl.ds(j * 16, 16)] = zivec

        r0 = s * ZR

        @pl.loop(0, ZR // _CE)
        def _zacc(j):
            pltpu.sync_copy(wbA, p_sh.at[pl.ds(r0 + j * _CE, _CE)])
            pltpu.sync_copy(exbA, d_sh.at[pl.ds(r0 + j * _CE, _CE)])

        zrem = ZR % _CE
        if zrem:
            zoff = r0 + (ZR // _CE) * _CE
            pltpu.sync_copy(wbA.at[pl.ds(0, zrem)], p_sh.at[pl.ds(zoff, zrem)])
            pltpu.sync_copy(exbA.at[pl.ds(0, zrem)], d_sh.at[pl.ds(zoff, zrem)])

        plsc.subcore_barrier()

        base = wid * n_chunks * _CE

        def load_and_fire(off, sidx, didx, ar, ad, fr, sem):
            pltpu.sync_copy(src_hbm.at[pl.ds(off, _CE)], sidx)
            pltpu.sync_copy(dst_hbm.at[pl.ds(off, _CE)], didx)
            pltpu.async_copy(asrc_hbm.at[sidx], ar, sem)
            pltpu.async_copy(adst_hbm.at[didx], ad, sem)
            pltpu.async_copy(feat_hbm.at[sidx], fr, sem)

        def wait_gathers(sidx, didx, ar, ad, fr, sem):
            pltpu.make_async_copy(asrc_hbm.at[sidx], ar, sem).wait()
            pltpu.make_async_copy(adst_hbm.at[didx], ad, sem).wait()
            pltpu.make_async_copy(feat_hbm.at[sidx], fr, sem).wait()

        def compute(ar, ad, fr, exb, wb):
            @pl.loop(0, _CE)
            def _edge(i):
                e = ar[i, :] + ad[i, :]
                e = jnp.where(e >= 0.0, e, _ALPHA * e)
                ex_vec = jnp.exp(e)
                exb[i, :] = ex_vec
                for b in range(B):
                    hd = (b * 16) // cpb
                    sc = ex_vec[hd]
                    wb[i, b * 16:(b + 1) * 16] = sc * fr[i, b * 16:(b + 1) * 16]

        def fire_scatters(off, wb, exb, didxS, ssem):
            pltpu.sync_copy(dst_hbm.at[pl.ds(off, _CE)], didxS)
            pltpu.async_copy(wb, p_sh.at[didxS], ssem, add=True)
            pltpu.async_copy(exb, d_sh.at[didxS], ssem, add=True)

        def wait_scatters(wb, exb, didxS, ssem):
            pltpu.make_async_copy(wb, p_sh.at[didxS], ssem).wait()
            pltpu.make_async_copy(exb, d_sh.at[didxS], ssem).wait()

        gbufA = (sidxA, didxA, arA, adA, frA)
        gbufB = (sidxB, didxB, arB, adB, frB)

        pltpu.async_copy(wbA, p_sh.at[didxSA], ssemA, add=True)
        pltpu.async_copy(exbA, d_sh.at[didxSA], ssemA, add=True)
        pltpu.async_copy(wbB, p_sh.at[didxSB], ssemB, add=True)
        pltpu.async_copy(exbB, d_sh.at[didxSB], ssemB, add=True)
        load_and_fire(base, *gbufA, gsemA)

        @pl.loop(0, n_chunks // 2)
        def _pair(q):
            off = base + q * (2 * _CE)
            wait_scatters(wbB, exbB, didxSB, ssemB)
            load_and_fire(off + _CE, *gbufB, gsemB)
            wait_gathers(*gbufA, gsemA)
            wait_scatters(wbA, exbA, didxSA, ssemA)
            compute(arA, adA, frA, exbA, wbA)
            fire_scatters(off, wbA, exbA, didxSA, ssemA)
            load_and_fire(off + 2 * _CE, *gbufA, gsemA)
            wait_gathers(*gbufB, gsemB)
            compute(arB, adB, frB, exbB, wbB)
            fire_scatters(off + _CE, wbB, exbB, didxSB, ssemB)

        wait_gathers(*gbufA, gsemA)
        wait_scatters(wbA, exbA, didxSA, ssemA)
        wait_scatters(wbB, exbB, didxSB, ssemB)

        plsc.subcore_barrier()
        pltpu.sync_copy(p_sh.at[pl.ds(r0, ZR)], p_out.at[c, pl.ds(r0, ZR)])
        pltpu.sync_copy(d_sh.at[pl.ds(r0, ZR)], d_out.at[c, pl.ds(r0, ZR)])

    return pl.kernel(
        body,
        out_type=(
            jax.ShapeDtypeStruct((2, NPp, F), f32),
            jax.ShapeDtypeStruct((2, NPp, 16), f32),
        ),
        mesh=mesh,
        compiler_params=pltpu.CompilerParams(use_tc_tiling_on_sc=False),
        scratch_types=(
            [
                pltpu.VMEM((_CE,), jnp.int32),
                pltpu.VMEM((_CE,), jnp.int32),
                pltpu.VMEM((_CE, 16), f32),
                pltpu.VMEM((_CE, 16), f32),
                pltpu.VMEM((_CE, F), f32),
                pltpu.VMEM((_CE, 16), f32),
                pltpu.VMEM((_CE, F), f32),
                pltpu.VMEM((_CE,), jnp.int32),
            ] * 2
            + [
                pltpu.VMEM_SHARED((NPp, F), f32),
                pltpu.VMEM_SHARED((NPp, 16), f32),
                pltpu.SemaphoreType.DMA,
                pltpu.SemaphoreType.DMA,
                pltpu.SemaphoreType.DMA,
                pltpu.SemaphoreType.DMA,
            ]
        ),
    )


def _ceil_to(v, m):
    return ((v + m - 1) // m) * m


def kernel(x, k_edges, W, a, W_out, a_out):
    n, nfeat = x.shape
    nheads, _, nhid = W.shape
    nclass = W_out.shape[1]
    K, _, E = k_edges.shape
    F1 = nheads * nhid

    NPp = _ceil_to(n, 32)
    if NPp == n:
        NPp += 32
    Ep = _ceil_to(E, _NW * _CE * 2)

    xp = jnp.zeros((NPp, nfeat), f32).at[:n].set(x)
    W_cat = W.transpose(1, 0, 2).reshape(nfeat, F1)
    rows = jnp.arange(F1)
    A_src = jnp.zeros((F1, 16), f32).at[rows, rows // nhid].set(
        a[:, :nhid].reshape(-1))
    A_dst = jnp.zeros((F1, 16), f32).at[rows, rows // nhid].set(
        a[:, nhid:].reshape(-1))
    Exp1 = jnp.zeros((16, F1), f32).at[rows // nhid, rows].set(1.0)
    A2_src = jnp.zeros((nclass, 16), f32).at[:, 0].set(a_out[:nclass])
    A2_dst = jnp.zeros((nclass, 16), f32).at[:, 0].set(a_out[nclass:])
    Exp2 = jnp.zeros((16, nclass), f32).at[0, :].set(1.0)

    pad = Ep + _CE - E
    srcp = jnp.concatenate(
        [k_edges[:, 0, :], jnp.zeros((K, pad), jnp.int32)], axis=1)
    dstp = jnp.concatenate(
        [k_edges[:, 1, :], jnp.full((K, pad), NPp - 1, jnp.int32)], axis=1)

    edge1 = _make_edge_pass(NPp, F1, nheads, Ep)
    edge2 = _make_edge_pass(NPp, nclass, 1, Ep)

    h1, s1, d1 = _prep(xp, W_cat, A_src, A_dst)
    feat = h1
    for l in range(K):
        p, dd = edge1(srcp[l], dstp[l], s1, d1, feat)
        feat = _merge(p, dd, h1, Exp1, apply_elu=(l == K - 1))

    h2, s2, d2 = _prep(feat, W_out, A2_src, A2_dst)
    feat2 = h2
    for l in range(K):
        p, dd = edge2(srcp[l], dstp[l], s2, d2, feat2)
        feat2 = _merge(p, dd, h2, Exp2, apply_elu=(l == K - 1))

    return _log_softmax(feat2)[:n]


# speedup vs baseline: 1.5942x; 1.5942x over previous
import functools

import jax
import jax.numpy as jnp
from jax import lax
from jax.experimental import pallas as pl
from jax.experimental.pallas import tpu as pltpu
from jax.experimental.pallas import tpu_sc as plsc

_ALPHA = 0.2
_BETA = 0.1
_CE = 112
_NW = 32

f32 = jnp.float32


def _prep(xp, W_cat, A_src, A_dst):
    NPp, KD = xp.shape
    FD = W_cat.shape[1]
    BN = 1024

    def body(x_ref, w_ref, as_ref, ad_ref, h_ref, s_ref, d_ref):
        h = jnp.dot(x_ref[...], w_ref[...], preferred_element_type=f32)
        h_ref[...] = h
        s_ref[...] = jnp.dot(h, as_ref[...], preferred_element_type=f32)
        d_ref[...] = jnp.dot(h, ad_ref[...], preferred_element_type=f32)

    return pl.pallas_call(
        body,
        grid=(pl.cdiv(NPp, BN),),
        in_specs=[
            pl.BlockSpec((BN, KD), lambda i: (i, 0)),
            pl.BlockSpec((KD, FD), lambda i: (0, 0)),
            pl.BlockSpec((FD, 16), lambda i: (0, 0)),
            pl.BlockSpec((FD, 16), lambda i: (0, 0)),
        ],
        out_specs=[
            pl.BlockSpec((BN, FD), lambda i: (i, 0)),
            pl.BlockSpec((BN, 16), lambda i: (i, 0)),
            pl.BlockSpec((BN, 16), lambda i: (i, 0)),
        ],
        out_shape=[
            jax.ShapeDtypeStruct((NPp, FD), f32),
            jax.ShapeDtypeStruct((NPp, 16), f32),
            jax.ShapeDtypeStruct((NPp, 16), f32),
        ],
    )(xp, W_cat, A_src, A_dst)


def _merge(p, d, h, expand, apply_elu):
    NPp, F = h.shape
    BN = 512

    def body(p_ref, d_ref, h_ref, e_ref, o_ref):
        den = d_ref[0] + d_ref[1] + 1e-16
        inv = jnp.dot(1.0 / den, e_ref[...], preferred_element_type=f32)
        v = (1.0 - _BETA) * (p_ref[0] + p_ref[1]) * inv + _BETA * h_ref[...]
        if apply_elu:
            v = jnp.where(v > 0, v, jnp.exp(v) - 1.0)
        o_ref[...] = v

    return pl.pallas_call(
        body,
        grid=(pl.cdiv(NPp, BN),),
        in_specs=[
            pl.BlockSpec((2, BN, F), lambda i: (0, i, 0)),
            pl.BlockSpec((2, BN, 16), lambda i: (0, i, 0)),
            pl.BlockSpec((BN, F), lambda i: (i, 0)),
            pl.BlockSpec((16, F), lambda i: (0, 0)),
        ],
        out_specs=pl.BlockSpec((BN, F), lambda i: (i, 0)),
        out_shape=jax.ShapeDtypeStruct((NPp, F), f32),
    )(p, d, h, expand)


def _log_softmax(x):
    NPp, C = x.shape
    BN = 2048

    def body(x_ref, o_ref):
        v = x_ref[...]
        m = jnp.max(v, axis=1, keepdims=True)
        ex = jnp.exp(v - m)
        s = jnp.sum(ex, axis=1, keepdims=True)
        o_ref[...] = v - m - jnp.log(s)

    return pl.pallas_call(
        body,
        grid=(pl.cdiv(NPp, BN),),
        in_specs=[pl.BlockSpec((BN, C), lambda i: (i, 0))],
        out_specs=pl.BlockSpec((BN, C), lambda i: (i, 0)),
        out_shape=jax.ShapeDtypeStruct((NPp, C), f32),
    )(x)


def _make_edge_pass(NPp, F, H, Ep):
    B = F // 16
    cpb = F // H
    ZR = NPp // 16
    n_chunks = Ep // (_NW * _CE)
    mesh = plsc.VectorSubcoreMesh(
        core_axis_name="c", subcore_axis_name="s", num_cores=2, num_subcores=16
    )

    def body(src_hbm, dst_hbm, asrc_hbm, adst_hbm, feat_hbm, p_out, d_out,
             sidxA, didxA, arA, adA, frA, exbA,
             sidxB, didxB, arB, adB, frB, exbB,
             p_sh, d_sh, gsemA, gsemB, ssem):
        c = lax.axis_index("c")
        s = lax.axis_index("s")
        wid = s * 2 + c

        zvec = jnp.zeros((16,), f32)

        @pl.loop(0, _CE)
        def _zbuf(r):
            for b in range(B):
                frA[r, b * 16:(b + 1) * 16] = zvec
            exbA[r, :] = zvec

        r0 = s * ZR

        @pl.loop(0, ZR // _CE)
        def _zacc(j):
            pltpu.sync_copy(frA, p_sh.at[pl.ds(r0 + j * _CE, _CE)])
            pltpu.sync_copy(exbA, d_sh.at[pl.ds(r0 + j * _CE, _CE)])

        zrem = ZR % _CE
        if zrem:
            zoff = r0 + (ZR // _CE) * _CE
            pltpu.sync_copy(frA.at[pl.ds(0, zrem)], p_sh.at[pl.ds(zoff, zrem)])
            pltpu.sync_copy(exbA.at[pl.ds(0, zrem)], d_sh.at[pl.ds(zoff, zrem)])

        plsc.subcore_barrier()

        base = wid * n_chunks * _CE

        def load_and_fire(off, sidx, didx, ar, ad, fr, sem):
            pltpu.sync_copy(src_hbm.at[pl.ds(off, _CE)], sidx)
            pltpu.sync_copy(dst_hbm.at[pl.ds(off, _CE)], didx)
            pltpu.async_copy(asrc_hbm.at[sidx], ar, sem)
            pltpu.async_copy(adst_hbm.at[didx], ad, sem)
            pltpu.async_copy(feat_hbm.at[sidx], fr, sem)

        def wait_gathers(sidx, didx, ar, ad, fr, sem):
            pltpu.make_async_copy(asrc_hbm.at[sidx], ar, sem).wait()
            pltpu.make_async_copy(adst_hbm.at[didx], ad, sem).wait()
            pltpu.make_async_copy(feat_hbm.at[sidx], fr, sem).wait()

        def compute(ar, ad, fr, exb):
            @pl.loop(0, _CE)
            def _edge(i):
                e = ar[i, :] + ad[i, :]
                e = jnp.where(e >= 0.0, e, _ALPHA * e)
                ex_vec = jnp.exp(e)
                exb[i, :] = ex_vec
                for b in range(B):
                    hd = (b * 16) // cpb
                    sc = ex_vec[hd]
                    fr[i, b * 16:(b + 1) * 16] = sc * fr[i, b * 16:(b + 1) * 16]

        def scatter_sync(fr, exb, didx):
            pltpu.async_copy(fr, p_sh.at[didx], ssem, add=True)
            pltpu.async_copy(exb, d_sh.at[didx], ssem, add=True)
            pltpu.make_async_copy(fr, p_sh.at[didx], ssem).wait()
            pltpu.make_async_copy(exb, d_sh.at[didx], ssem).wait()

        gbufA = (sidxA, didxA, arA, adA, frA)
        gbufB = (sidxB, didxB, arB, adB, frB)

        load_and_fire(base, *gbufA, gsemA)

        @pl.loop(0, n_chunks // 2)
        def _pair(q):
            off = base + q * (2 * _CE)
            load_and_fire(off + _CE, *gbufB, gsemB)
            wait_gathers(*gbufA, gsemA)
            compute(arA, adA, frA, exbA)
            scatter_sync(frA, exbA, didxA)
            load_and_fire(off + 2 * _CE, *gbufA, gsemA)
            wait_gathers(*gbufB, gsemB)
            compute(arB, adB, frB, exbB)
            scatter_sync(frB, exbB, didxB)

        wait_gathers(*gbufA, gsemA)

        plsc.subcore_barrier()
        pltpu.sync_copy(p_sh.at[pl.ds(r0, ZR)], p_out.at[c, pl.ds(r0, ZR)])
        pltpu.sync_copy(d_sh.at[pl.ds(r0, ZR)], d_out.at[c, pl.ds(r0, ZR)])

    return pl.kernel(
        body,
        out_type=(
            jax.ShapeDtypeStruct((2, NPp, F), f32),
            jax.ShapeDtypeStruct((2, NPp, 16), f32),
        ),
        mesh=mesh,
        compiler_params=pltpu.CompilerParams(use_tc_tiling_on_sc=False),
        scratch_types=(
            [
                pltpu.VMEM((_CE,), jnp.int32),
                pltpu.VMEM((_CE,), jnp.int32),
                pltpu.VMEM((_CE, 16), f32),
                pltpu.VMEM((_CE, 16), f32),
                pltpu.VMEM((_CE, F), f32),
                pltpu.VMEM((_CE, 16), f32),
            ] * 2
            + [
                pltpu.VMEM_SHARED((NPp, F), f32),
                pltpu.VMEM_SHARED((NPp, 16), f32),
                pltpu.SemaphoreType.DMA,
                pltpu.SemaphoreType.DMA,
                pltpu.SemaphoreType.DMA,
            ]
        ),
    )


def _ceil_to(v, m):
    return ((v + m - 1) // m) * m


def kernel(x, k_edges, W, a, W_out, a_out):
    n, nfeat = x.shape
    nheads, _, nhid = W.shape
    nclass = W_out.shape[1]
    K, _, E = k_edges.shape
    F1 = nheads * nhid

    NPp = _ceil_to(n, 32)
    if NPp == n:
        NPp += 32
    Ep = _ceil_to(E, _NW * _CE * 2)

    xp = jnp.zeros((NPp, nfeat), f32).at[:n].set(x)
    W_cat = W.transpose(1, 0, 2).reshape(nfeat, F1)
    rows = jnp.arange(F1)
    A_src = jnp.zeros((F1, 16), f32).at[rows, rows // nhid].set(
        a[:, :nhid].reshape(-1))
    A_dst = jnp.zeros((F1, 16), f32).at[rows, rows // nhid].set(
        a[:, nhid:].reshape(-1))
    Exp1 = jnp.zeros((16, F1), f32).at[rows // nhid, rows].set(1.0)
    A2_src = jnp.zeros((nclass, 16), f32).at[:, 0].set(a_out[:nclass])
    A2_dst = jnp.zeros((nclass, 16), f32).at[:, 0].set(a_out[nclass:])
    Exp2 = jnp.zeros((16, nclass), f32).at[0, :].set(1.0)

    pad = Ep + _CE - E
    srcp = jnp.concatenate(
        [k_edges[:, 0, :], jnp.zeros((K, pad), jnp.int32)], axis=1)
    dstp = jnp.concatenate(
        [k_edges[:, 1, :], jnp.full((K, pad), NPp - 1, jnp.int32)], axis=1)

    edge1 = _make_edge_pass(NPp, F1, nheads, Ep)
    edge2 = _make_edge_pass(NPp, nclass, 1, Ep)

    h1, s1, d1 = _prep(xp, W_cat, A_src, A_dst)
    feat = h1
    for l in range(K):
        p, dd = edge1(srcp[l], dstp[l], s1, d1, feat)
        feat = _merge(p, dd, h1, Exp1, apply_elu=(l == K - 1))

    h2, s2, d2 = _prep(feat, W_out, A2_src, A2_dst)
    feat2 = h2
    for l in range(K):
        p, dd = edge2(srcp[l], dstp[l], s2, d2, feat2)
        feat2 = _merge(p, dd, h2, Exp2, apply_elu=(l == K - 1))

    return _log_softmax(feat2)[:n]


# speedup vs baseline: 1.6828x; 1.0556x over previous
import functools

import jax
import jax.numpy as jnp
from jax import lax
from jax.experimental import pallas as pl
from jax.experimental.pallas import tpu as pltpu
from jax.experimental.pallas import tpu_sc as plsc

_ALPHA = 0.2
_BETA = 0.1
_CE = 112
_NW = 32

f32 = jnp.float32


def _prep(xp, W_cat, A_src, A_dst):
    NPp, KD = xp.shape
    FD = W_cat.shape[1]
    BN = 1024

    def body(x_ref, w_ref, as_ref, ad_ref, hx_ref, d_ref):
        h = jnp.dot(x_ref[...], w_ref[...], preferred_element_type=f32)
        s = jnp.dot(h, as_ref[...], preferred_element_type=f32)
        hx_ref[...] = jnp.concatenate([h, s], axis=1)
        d_ref[...] = jnp.dot(h, ad_ref[...], preferred_element_type=f32)

    return pl.pallas_call(
        body,
        grid=(pl.cdiv(NPp, BN),),
        in_specs=[
            pl.BlockSpec((BN, KD), lambda i: (i, 0)),
            pl.BlockSpec((KD, FD), lambda i: (0, 0)),
            pl.BlockSpec((FD, 16), lambda i: (0, 0)),
            pl.BlockSpec((FD, 16), lambda i: (0, 0)),
        ],
        out_specs=[
            pl.BlockSpec((BN, FD + 16), lambda i: (i, 0)),
            pl.BlockSpec((BN, 16), lambda i: (i, 0)),
        ],
        out_shape=[
            jax.ShapeDtypeStruct((NPp, FD + 16), f32),
            jax.ShapeDtypeStruct((NPp, 16), f32),
        ],
    )(xp, W_cat, A_src, A_dst)


def _merge(p2, hx, expand, apply_elu):
    NPp, Fx = hx.shape
    F = Fx - 16
    BN = 512

    def body(p_ref, hx_ref, e_ref, o_ref):
        den = p_ref[0, :, F:] + p_ref[1, :, F:] + 1e-16
        inv = jnp.dot(1.0 / den, e_ref[...], preferred_element_type=f32)
        v = ((1.0 - _BETA) * (p_ref[0, :, :F] + p_ref[1, :, :F]) * inv
             + _BETA * hx_ref[:, :F])
        if apply_elu:
            v = jnp.where(v > 0, v, jnp.exp(v) - 1.0)
        o_ref[...] = jnp.concatenate([v, hx_ref[:, F:]], axis=1)

    return pl.pallas_call(
        body,
        grid=(pl.cdiv(NPp, BN),),
        in_specs=[
            pl.BlockSpec((2, BN, Fx), lambda i: (0, i, 0)),
            pl.BlockSpec((BN, Fx), lambda i: (i, 0)),
            pl.BlockSpec((16, F), lambda i: (0, 0)),
        ],
        out_specs=pl.BlockSpec((BN, Fx), lambda i: (i, 0)),
        out_shape=jax.ShapeDtypeStruct((NPp, Fx), f32),
    )(p2, hx, expand)


def _log_softmax(x):
    NPp, C = x.shape
    BN = 2048

    def body(x_ref, o_ref):
        v = x_ref[...]
        m = jnp.max(v, axis=1, keepdims=True)
        ex = jnp.exp(v - m)
        s = jnp.sum(ex, axis=1, keepdims=True)
        o_ref[...] = v - m - jnp.log(s)

    return pl.pallas_call(
        body,
        grid=(pl.cdiv(NPp, BN),),
        in_specs=[pl.BlockSpec((BN, C), lambda i: (i, 0))],
        out_specs=pl.BlockSpec((BN, C), lambda i: (i, 0)),
        out_shape=jax.ShapeDtypeStruct((NPp, C), f32),
    )(x)


def _make_edge_pass(NPp, F, H, Ep):
    B = F // 16
    cpb = F // H
    Fx = F + 16
    BX = Fx // 16
    ZR = NPp // 16
    n_chunks = Ep // (_NW * _CE)
    mesh = plsc.VectorSubcoreMesh(
        core_axis_name="c", subcore_axis_name="s", num_cores=2, num_subcores=16
    )

    def body(src_hbm, dst_hbm, adst_hbm, featx_hbm, p_out,
             sidxA, didxA, adA, frxA,
             sidxB, didxB, adB, frxB,
             px_sh, gsemA, gsemB, ssem):
        c = lax.axis_index("c")
        s = lax.axis_index("s")
        wid = s * 2 + c

        zvec = jnp.zeros((16,), f32)

        @pl.loop(0, _CE)
        def _zbuf(r):
            for b in range(BX):
                frxA[r, b * 16:(b + 1) * 16] = zvec

        r0 = s * ZR

        @pl.loop(0, ZR // _CE)
        def _zacc(j):
            pltpu.sync_copy(frxA, px_sh.at[pl.ds(r0 + j * _CE, _CE)])

        zrem = ZR % _CE
        if zrem:
            zoff = r0 + (ZR // _CE) * _CE
            pltpu.sync_copy(frxA.at[pl.ds(0, zrem)], px_sh.at[pl.ds(zoff, zrem)])

        plsc.subcore_barrier()

        base = wid * n_chunks * _CE

        def load_and_fire(off, sidx, didx, ad, frx, sem):
            pltpu.sync_copy(src_hbm.at[pl.ds(off, _CE)], sidx)
            pltpu.sync_copy(dst_hbm.at[pl.ds(off, _CE)], didx)
            pltpu.async_copy(featx_hbm.at[sidx], frx, sem)
            pltpu.async_copy(adst_hbm.at[didx], ad, sem)

        def wait_gathers(sidx, didx, ad, frx, sem):
            pltpu.make_async_copy(featx_hbm.at[sidx], frx, sem).wait()
            pltpu.make_async_copy(adst_hbm.at[didx], ad, sem).wait()

        def compute(ad, frx):
            @pl.loop(0, _CE)
            def _edge(i):
                e = frx[i, F:Fx] + ad[i, :]
                e = jnp.where(e >= 0.0, e, _ALPHA * e)
                ex_vec = jnp.exp(e)
                for b in range(B):
                    hd = (b * 16) // cpb
                    sc = ex_vec[hd]
                    frx[i, b * 16:(b + 1) * 16] = sc * frx[i, b * 16:(b + 1) * 16]
                frx[i, F:Fx] = ex_vec

        def scatter_sync(frx, didx):
            pltpu.async_copy(frx, px_sh.at[didx], ssem, add=True)
            pltpu.make_async_copy(frx, px_sh.at[didx], ssem).wait()

        gbufA = (sidxA, didxA, adA, frxA)
        gbufB = (sidxB, didxB, adB, frxB)

        load_and_fire(base, *gbufA, gsemA)

        @pl.loop(0, n_chunks // 2)
        def _pair(q):
            off = base + q * (2 * _CE)
            load_and_fire(off + _CE, *gbufB, gsemB)
            wait_gathers(*gbufA, gsemA)
            compute(adA, frxA)
            scatter_sync(frxA, didxA)
            load_and_fire(off + 2 * _CE, *gbufA, gsemA)
            wait_gathers(*gbufB, gsemB)
            compute(adB, frxB)
            scatter_sync(frxB, didxB)

        wait_gathers(*gbufA, gsemA)

        plsc.subcore_barrier()
        pltpu.sync_copy(px_sh.at[pl.ds(r0, ZR)], p_out.at[c, pl.ds(r0, ZR)])

    return pl.kernel(
        body,
        out_type=jax.ShapeDtypeStruct((2, NPp, Fx), f32),
        mesh=mesh,
        compiler_params=pltpu.CompilerParams(use_tc_tiling_on_sc=False),
        scratch_types=(
            [
                pltpu.VMEM((_CE,), jnp.int32),
                pltpu.VMEM((_CE,), jnp.int32),
                pltpu.VMEM((_CE, 16), f32),
                pltpu.VMEM((_CE, Fx), f32),
            ] * 2
            + [
                pltpu.VMEM_SHARED((NPp, Fx), f32),
                pltpu.SemaphoreType.DMA,
                pltpu.SemaphoreType.DMA,
                pltpu.SemaphoreType.DMA,
            ]
        ),
    )


def _ceil_to(v, m):
    return ((v + m - 1) // m) * m


def kernel(x, k_edges, W, a, W_out, a_out):
    n, nfeat = x.shape
    nheads, _, nhid = W.shape
    nclass = W_out.shape[1]
    K, _, E = k_edges.shape
    F1 = nheads * nhid

    NPp = _ceil_to(n, 32)
    if NPp == n:
        NPp += 32
    Ep = _ceil_to(E, _NW * _CE * 2)

    xp = jnp.zeros((NPp, nfeat), f32).at[:n].set(x)
    W_cat = W.transpose(1, 0, 2).reshape(nfeat, F1)
    rows = jnp.arange(F1)
    A_src = jnp.zeros((F1, 16), f32).at[rows, rows // nhid].set(
        a[:, :nhid].reshape(-1))
    A_dst = jnp.zeros((F1, 16), f32).at[rows, rows // nhid].set(
        a[:, nhid:].reshape(-1))
    Exp1 = jnp.zeros((16, F1), f32).at[rows // nhid, rows].set(1.0)
    A2_src = jnp.zeros((nclass, 16), f32).at[:, 0].set(a_out[:nclass])
    A2_dst = jnp.zeros((nclass, 16), f32).at[:, 0].set(a_out[nclass:])
    Exp2 = jnp.zeros((16, nclass), f32).at[0, :].set(1.0)

    pad = Ep + _CE - E
    srcp = jnp.concatenate(
        [k_edges[:, 0, :], jnp.zeros((K, pad), jnp.int32)], axis=1)
    dstp = jnp.concatenate(
        [k_edges[:, 1, :], jnp.full((K, pad), NPp - 1, jnp.int32)], axis=1)

    edge1 = _make_edge_pass(NPp, F1, nheads, Ep)
    edge2 = _make_edge_pass(NPp, nclass, 1, Ep)

    h1x, d1 = _prep(xp, W_cat, A_src, A_dst)
    featx = h1x
    for l in range(K):
        p2 = edge1(srcp[l], dstp[l], d1, featx)
        featx = _merge(p2, h1x, Exp1, apply_elu=(l == K - 1))

    h2x, d2 = _prep(featx[:, :F1], W_out, A2_src, A2_dst)
    featx2 = h2x
    for l in range(K):
        p2 = edge2(srcp[l], dstp[l], d2, featx2)
        featx2 = _merge(p2, h2x, Exp2, apply_elu=(l == K - 1))

    return _log_softmax(featx2[:, :nclass])[:n]


# speedup vs baseline: 1.7024x; 1.0116x over previous
import functools

import jax
import jax.numpy as jnp
from jax import lax
from jax.experimental import pallas as pl
from jax.experimental.pallas import tpu as pltpu
from jax.experimental.pallas import tpu_sc as plsc

_ALPHA = 0.2
_BETA = 0.1
_CE = 120
_NW = 32

f32 = jnp.float32


def _prep(xp, W_cat, A_src, A_dst):
    NPp, KD = xp.shape
    FD = W_cat.shape[1]
    BN = 1024

    def body(x_ref, w_ref, as_ref, ad_ref, hx_ref, d_ref):
        h = jnp.dot(x_ref[...], w_ref[...], preferred_element_type=f32)
        s = jnp.dot(h, as_ref[...], preferred_element_type=f32)
        hx_ref[...] = jnp.concatenate([h, s], axis=1)
        d_ref[...] = jnp.dot(h, ad_ref[...], preferred_element_type=f32)

    return pl.pallas_call(
        body,
        grid=(pl.cdiv(NPp, BN),),
        in_specs=[
            pl.BlockSpec((BN, KD), lambda i: (i, 0)),
            pl.BlockSpec((KD, FD), lambda i: (0, 0)),
            pl.BlockSpec((FD, 16), lambda i: (0, 0)),
            pl.BlockSpec((FD, 16), lambda i: (0, 0)),
        ],
        out_specs=[
            pl.BlockSpec((BN, FD + 16), lambda i: (i, 0)),
            pl.BlockSpec((BN, 16), lambda i: (i, 0)),
        ],
        out_shape=[
            jax.ShapeDtypeStruct((NPp, FD + 16), f32),
            jax.ShapeDtypeStruct((NPp, 16), f32),
        ],
    )(xp, W_cat, A_src, A_dst)


def _merge(p2, hx, expand, apply_elu):
    NPp, Fx = hx.shape
    F = Fx - 16
    BN = 512

    def body(p_ref, hx_ref, e_ref, o_ref):
        den = p_ref[0, :, F:] + p_ref[1, :, F:] + 1e-16
        inv = jnp.dot(1.0 / den, e_ref[...], preferred_element_type=f32)
        v = ((1.0 - _BETA) * (p_ref[0, :, :F] + p_ref[1, :, :F]) * inv
             + _BETA * hx_ref[:, :F])
        if apply_elu:
            v = jnp.where(v > 0, v, jnp.exp(v) - 1.0)
        o_ref[...] = jnp.concatenate([v, hx_ref[:, F:]], axis=1)

    return pl.pallas_call(
        body,
        grid=(pl.cdiv(NPp, BN),),
        in_specs=[
            pl.BlockSpec((2, BN, Fx), lambda i: (0, i, 0)),
            pl.BlockSpec((BN, Fx), lambda i: (i, 0)),
            pl.BlockSpec((16, F), lambda i: (0, 0)),
        ],
        out_specs=pl.BlockSpec((BN, Fx), lambda i: (i, 0)),
        out_shape=jax.ShapeDtypeStruct((NPp, Fx), f32),
    )(p2, hx, expand)


def _log_softmax(x):
    NPp, C = x.shape
    BN = 2048

    def body(x_ref, o_ref):
        v = x_ref[...]
        m = jnp.max(v, axis=1, keepdims=True)
        ex = jnp.exp(v - m)
        s = jnp.sum(ex, axis=1, keepdims=True)
        o_ref[...] = v - m - jnp.log(s)

    return pl.pallas_call(
        body,
        grid=(pl.cdiv(NPp, BN),),
        in_specs=[pl.BlockSpec((BN, C), lambda i: (i, 0))],
        out_specs=pl.BlockSpec((BN, C), lambda i: (i, 0)),
        out_shape=jax.ShapeDtypeStruct((NPp, C), f32),
    )(x)


def _make_edge_pass(NPp, F, H, Ep):
    B = F // 16
    cpb = F // H
    Fx = F + 16
    BX = Fx // 16
    ZR = NPp // 16
    n_chunks = Ep // (_NW * _CE)
    mesh = plsc.VectorSubcoreMesh(
        core_axis_name="c", subcore_axis_name="s", num_cores=2, num_subcores=16
    )

    def body(src_hbm, dst_hbm, adst_hbm, featx_hbm, p_out,
             sidxA, didxA, adA, frxA,
             sidxB, didxB, adB, frxB,
             px_sh, gsemA, gsemB, ssem):
        c = lax.axis_index("c")
        s = lax.axis_index("s")
        wid = s * 2 + c

        zvec = jnp.zeros((16,), f32)

        @pl.loop(0, _CE)
        def _zbuf(r):
            for b in range(BX):
                frxA[r, b * 16:(b + 1) * 16] = zvec

        r0 = s * ZR

        @pl.loop(0, ZR // _CE)
        def _zacc(j):
            pltpu.sync_copy(frxA, px_sh.at[pl.ds(r0 + j * _CE, _CE)])

        zrem = ZR % _CE
        if zrem:
            zoff = r0 + (ZR // _CE) * _CE
            pltpu.sync_copy(frxA.at[pl.ds(0, zrem)], px_sh.at[pl.ds(zoff, zrem)])

        plsc.subcore_barrier()

        base = wid * n_chunks * _CE

        def load_and_fire(off, sidx, didx, ad, frx, sem):
            pltpu.sync_copy(src_hbm.at[pl.ds(off, _CE)], sidx)
            pltpu.sync_copy(dst_hbm.at[pl.ds(off, _CE)], didx)
            pltpu.async_copy(featx_hbm.at[sidx], frx, sem)
            pltpu.async_copy(adst_hbm.at[didx], ad, sem)

        def wait_gathers(sidx, didx, ad, frx, sem):
            pltpu.make_async_copy(featx_hbm.at[sidx], frx, sem).wait()
            pltpu.make_async_copy(adst_hbm.at[didx], ad, sem).wait()

        def compute(ad, frx):
            @pl.loop(0, _CE)
            def _edge(i):
                e = frx[i, F:Fx] + ad[i, :]
                e = jnp.where(e >= 0.0, e, _ALPHA * e)
                ex_vec = jnp.exp(e)
                for b in range(B):
                    hd = (b * 16) // cpb
                    sc = ex_vec[hd]
                    frx[i, b * 16:(b + 1) * 16] = sc * frx[i, b * 16:(b + 1) * 16]
                frx[i, F:Fx] = ex_vec

        def scatter_sync(frx, didx):
            pltpu.async_copy(frx, px_sh.at[didx], ssem, add=True)
            pltpu.make_async_copy(frx, px_sh.at[didx], ssem).wait()

        gbufA = (sidxA, didxA, adA, frxA)
        gbufB = (sidxB, didxB, adB, frxB)

        load_and_fire(base, *gbufA, gsemA)

        @pl.loop(0, n_chunks // 2)
        def _pair(q):
            off = base + q * (2 * _CE)
            load_and_fire(off + _CE, *gbufB, gsemB)
            wait_gathers(*gbufA, gsemA)
            compute(adA, frxA)
            scatter_sync(frxA, didxA)
            load_and_fire(off + 2 * _CE, *gbufA, gsemA)
            wait_gathers(*gbufB, gsemB)
            compute(adB, frxB)
            scatter_sync(frxB, didxB)

        wait_gathers(*gbufA, gsemA)

        plsc.subcore_barrier()
        pltpu.sync_copy(px_sh.at[pl.ds(r0, ZR)], p_out.at[c, pl.ds(r0, ZR)])

    return pl.kernel(
        body,
        out_type=jax.ShapeDtypeStruct((2, NPp, Fx), f32),
        mesh=mesh,
        compiler_params=pltpu.CompilerParams(use_tc_tiling_on_sc=False),
        scratch_types=(
            [
                pltpu.VMEM((_CE,), jnp.int32),
                pltpu.VMEM((_CE,), jnp.int32),
                pltpu.VMEM((_CE, 16), f32),
                pltpu.VMEM((_CE, Fx), f32),
            ] * 2
            + [
                pltpu.VMEM_SHARED((NPp, Fx), f32),
                pltpu.SemaphoreType.DMA,
                pltpu.SemaphoreType.DMA,
                pltpu.SemaphoreType.DMA,
            ]
        ),
    )


def _ceil_to(v, m):
    return ((v + m - 1) // m) * m


def kernel(x, k_edges, W, a, W_out, a_out):
    n, nfeat = x.shape
    nheads, _, nhid = W.shape
    nclass = W_out.shape[1]
    K, _, E = k_edges.shape
    F1 = nheads * nhid

    NPp = _ceil_to(n, 32)
    if NPp == n:
        NPp += 32
    Ep = _ceil_to(E, _NW * _CE * 2)

    xp = jnp.zeros((NPp, nfeat), f32).at[:n].set(x)
    W_cat = W.transpose(1, 0, 2).reshape(nfeat, F1)
    rows = jnp.arange(F1)
    A_src = jnp.zeros((F1, 16), f32).at[rows, rows // nhid].set(
        a[:, :nhid].reshape(-1))
    A_dst = jnp.zeros((F1, 16), f32).at[rows, rows // nhid].set(
        a[:, nhid:].reshape(-1))
    Exp1 = jnp.zeros((16, F1), f32).at[rows // nhid, rows].set(1.0)
    A2_src = jnp.zeros((nclass, 16), f32).at[:, 0].set(a_out[:nclass])
    A2_dst = jnp.zeros((nclass, 16), f32).at[:, 0].set(a_out[nclass:])
    Exp2 = jnp.zeros((16, nclass), f32).at[0, :].set(1.0)

    pad = Ep + _CE - E
    srcp = jnp.concatenate(
        [k_edges[:, 0, :], jnp.zeros((K, pad), jnp.int32)], axis=1)
    dstp = jnp.concatenate(
        [k_edges[:, 1, :], jnp.full((K, pad), NPp - 1, jnp.int32)], axis=1)

    edge1 = _make_edge_pass(NPp, F1, nheads, Ep)
    edge2 = _make_edge_pass(NPp, nclass, 1, Ep)

    h1x, d1 = _prep(xp, W_cat, A_src, A_dst)
    featx = h1x
    for l in range(K):
        p2 = edge1(srcp[l], dstp[l], d1, featx)
        featx = _merge(p2, h1x, Exp1, apply_elu=(l == K - 1))

    h2x, d2 = _prep(featx[:, :F1], W_out, A2_src, A2_dst)
    featx2 = h2x
    for l in range(K):
        p2 = edge2(srcp[l], dstp[l], d2, featx2)
        featx2 = _merge(p2, h2x, Exp2, apply_elu=(l == K - 1))

    return _log_softmax(featx2[:, :nclass])[:n]
